# Initial kernel scaffold; baseline (speedup 1.0000x reference)
import jax
import jax.numpy as jnp
from jax.experimental import pallas as pl


def kernel(X, W, r):
    raise NotImplementedError("write your pallas kernel here")


# speedup vs baseline: 1.5196x; 1.5196x over previous
import jax
import jax.numpy as jnp
from jax.experimental import pallas as pl
from jax.experimental.pallas import tpu as pltpu

BN = 512


def _gmlvq_body(x_ref, w_ref, rc_ref, out_ref):
    rc = rc_ref[0, :]
    xb = x_ref[...]
    wb = w_ref[...]
    wr = wb * rc.astype(jnp.bfloat16)[None, :]
    cross = jax.lax.dot_general(
        xb, wr, (((1,), (1,)), ((), ())),
        preferred_element_type=jnp.float32)
    x32 = xb.astype(jnp.float32)
    w32 = wb.astype(jnp.float32)
    x2 = jnp.sum(x32 * x32 * rc[None, :], axis=1)
    w2 = jnp.sum(w32 * w32 * rc[None, :], axis=1)
    out_ref[...] = (x2[:, None] + w2[None, :]) - 2.0 * cross


def kernel(X, W, r):
    n, f = X.shape
    c, _ = W.shape
    rc = jnp.clip(r, 1e-6, None).reshape(1, f)
    xb = X.astype(jnp.bfloat16)
    wb = W.astype(jnp.bfloat16)
    return pl.pallas_call(
        _gmlvq_body,
        grid=(n // BN,),
        in_specs=[
            pl.BlockSpec((BN, f), lambda i: (i, 0)),
            pl.BlockSpec((c, f), lambda i: (0, 0)),
            pl.BlockSpec((1, f), lambda i: (0, 0)),
        ],
        out_specs=pl.BlockSpec((BN, c), lambda i: (i, 0)),
        out_shape=jax.ShapeDtypeStruct((n, c), jnp.float32),
    )(xb, wb, rc)


# speedup vs baseline: 1.5210x; 1.0009x over previous
import jax
import jax.numpy as jnp
from jax.experimental import pallas as pl
from jax.experimental.pallas import tpu as pltpu

BN = 512


def _gmlvq_body(x_ref, w_ref, rc_ref, out_ref):
    rc = rc_ref[0, :]
    xb = x_ref[...]
    wb = w_ref[...]
    wr = wb * rc.astype(jnp.bfloat16)[None, :]
    cross = jax.lax.dot_general(
        xb, wr, (((1,), (1,)), ((), ())),
        preferred_element_type=jnp.float32)
    x32 = xb.astype(jnp.float32)
    w32 = wb.astype(jnp.float32)
    x2 = jnp.sum(x32 * x32 * rc[None, :], axis=1)
    w2 = jnp.sum(w32 * w32 * rc[None, :], axis=1)
    out_ref[...] = (x2[:, None] + w2[None, :]) - 2.0 * cross


def kernel(X, W, r):
    n, f = X.shape
    c, _ = W.shape
    rc = jnp.clip(r, 1e-6, None).reshape(1, f)
    xb = X.astype(jnp.bfloat16)
    wb = W.astype(jnp.bfloat16)
    return pl.pallas_call(
        _gmlvq_body,
        grid=(n // BN,),
        in_specs=[
            pl.BlockSpec((BN, f), lambda i: (i, 0)),
            pl.BlockSpec((c, f), lambda i: (0, 0)),
            pl.BlockSpec((1, f), lambda i: (0, 0)),
        ],
        out_specs=pl.BlockSpec((BN, c), lambda i: (i, 0)),
        out_shape=jax.ShapeDtypeStruct((n, c), jnp.float32),
        compiler_params=pltpu.CompilerParams(
            dimension_semantics=("parallel",)),
    )(xb, wb, rc)


# speedup vs baseline: 1.5548x; 1.0222x over previous
import jax
import jax.numpy as jnp
from jax.experimental import pallas as pl
from jax.experimental.pallas import tpu as pltpu

BN = 512


def _gmlvq_body(x_ref, w_ref, rc_ref, out_ref, wr_ref, w2_ref):
    rc = rc_ref[0, :]

    @pl.when(pl.program_id(0) == 0)
    def _prep():
        wb = w_ref[...]
        wr_ref[...] = wb * (-2.0 * rc).astype(jnp.bfloat16)[None, :]
        w32 = wb.astype(jnp.float32)
        w2_ref[0, :] = jnp.sum(w32 * w32 * rc[None, :], axis=1)

    xb = x_ref[...]
    cross = jax.lax.dot_general(
        xb, wr_ref[...], (((1,), (1,)), ((), ())),
        preferred_element_type=jnp.float32)
    x32 = xb.astype(jnp.float32)
    x2 = jnp.sum(x32 * x32 * rc[None, :], axis=1)
    out_ref[...] = cross + x2[:, None] + w2_ref[0, :][None, :]


def kernel(X, W, r):
    n, f = X.shape
    c, _ = W.shape
    rc = jnp.clip(r, 1e-6, None).reshape(1, f)
    xb = X.astype(jnp.bfloat16)
    wb = W.astype(jnp.bfloat16)
    return pl.pallas_call(
        _gmlvq_body,
        grid=(n // BN,),
        in_specs=[
            pl.BlockSpec((BN, f), lambda i: (i, 0)),
            pl.BlockSpec((c, f), lambda i: (0, 0)),
            pl.BlockSpec((1, f), lambda i: (0, 0)),
        ],
        out_specs=pl.BlockSpec((BN, c), lambda i: (i, 0)),
        out_shape=jax.ShapeDtypeStruct((n, c), jnp.float32),
        scratch_shapes=[
            pltpu.VMEM((c, f), jnp.bfloat16),
            pltpu.VMEM((1, c), jnp.float32),
        ],
        compiler_params=pltpu.CompilerParams(
            dimension_semantics=("arbitrary",)),
    )(xb, wb, rc)


# speedup vs baseline: 1.7539x; 1.1280x over previous
import jax
import jax.numpy as jnp
from jax.experimental import pallas as pl
from jax.experimental.pallas import tpu as pltpu

BN = 1024


def _gmlvq_body(x_ref, w_ref, rc_ref, out_ref, wr_ref, w2_ref):
    rc = rc_ref[0, :]

    @pl.when(pl.program_id(0) == 0)
    def _prep():
        wb = w_ref[...]
        wr_ref[...] = wb * (-2.0 * rc).astype(jnp.bfloat16)[None, :]
        w32 = wb.astype(jnp.float32)
        w2_ref[0, :] = jnp.sum(w32 * w32 * rc[None, :], axis=1)

    xb = x_ref[...]
    cross = jax.lax.dot_general(
        xb, wr_ref[...], (((1,), (1,)), ((), ())),
        preferred_element_type=jnp.float32)
    x32 = xb.astype(jnp.float32)
    x2 = jnp.sum(x32 * x32 * rc[None, :], axis=1)
    out_ref[...] = cross + x2[:, None] + w2_ref[0, :][None, :]


def kernel(X, W, r):
    n, f = X.shape
    c, _ = W.shape
    rc = jnp.clip(r, 1e-6, None).reshape(1, f)
    xb = X.astype(jnp.bfloat16)
    wb = W.astype(jnp.bfloat16)
    return pl.pallas_call(
        _gmlvq_body,
        grid=(n // BN,),
        in_specs=[
            pl.BlockSpec((BN, f), lambda i: (i, 0)),
            pl.BlockSpec((c, f), lambda i: (0, 0)),
            pl.BlockSpec((1, f), lambda i: (0, 0)),
        ],
        out_specs=pl.BlockSpec((BN, c), lambda i: (i, 0)),
        out_shape=jax.ShapeDtypeStruct((n, c), jnp.float32),
        scratch_shapes=[
            pltpu.VMEM((c, f), jnp.bfloat16),
            pltpu.VMEM((1, c), jnp.float32),
        ],
        compiler_params=pltpu.CompilerParams(
            dimension_semantics=("arbitrary",)),
    )(xb, wb, rc)


# speedup vs baseline: 1.8295x; 1.0432x over previous
import jax
import jax.numpy as jnp
from jax.experimental import pallas as pl
from jax.experimental.pallas import tpu as pltpu

BN = 2048


def _gmlvq_body(x_ref, w_ref, rc_ref, out_ref, wr_ref, w2_ref):
    rc = rc_ref[0, :]

    @pl.when(pl.program_id(0) == 0)
    def _prep():
        wb = w_ref[...]
        wr_ref[...] = wb * (-2.0 * rc).astype(jnp.bfloat16)[None, :]
        w32 = wb.astype(jnp.float32)
        w2_ref[0, :] = jnp.sum(w32 * w32 * rc[None, :], axis=1)

    xb = x_ref[...]
    cross = jax.lax.dot_general(
        xb, wr_ref[...], (((1,), (1,)), ((), ())),
        preferred_element_type=jnp.float32)
    x32 = xb.astype(jnp.float32)
    x2 = jnp.sum(x32 * x32 * rc[None, :], axis=1)
    out_ref[...] = cross + x2[:, None] + w2_ref[0, :][None, :]


def kernel(X, W, r):
    n, f = X.shape
    c, _ = W.shape
    rc = jnp.clip(r, 1e-6, None).reshape(1, f)
    xb = X.astype(jnp.bfloat16)
    wb = W.astype(jnp.bfloat16)
    return pl.pallas_call(
        _gmlvq_body,
        grid=(n // BN,),
        in_specs=[
            pl.BlockSpec((BN, f), lambda i: (i, 0)),
            pl.BlockSpec((c, f), lambda i: (0, 0)),
            pl.BlockSpec((1, f), lambda i: (0, 0)),
        ],
        out_specs=pl.BlockSpec((BN, c), lambda i: (i, 0)),
        out_shape=jax.ShapeDtypeStruct((n, c), jnp.float32),
        scratch_shapes=[
            pltpu.VMEM((c, f), jnp.bfloat16),
            pltpu.VMEM((1, c), jnp.float32),
        ],
        compiler_params=pltpu.CompilerParams(
            dimension_semantics=("arbitrary",)),
    )(xb, wb, rc)


# speedup vs baseline: 2.6869x; 1.4686x over previous
import jax
import jax.numpy as jnp
from jax.experimental import pallas as pl
from jax.experimental.pallas import tpu as pltpu

BN = 2048


def _gmlvq_body(x_ref, w_ref, rc_ref, out_ref, wr_ref, w2_ref):
    rc = rc_ref[0, :]

    @pl.when(pl.program_id(0) == 0)
    def _prep():
        w32 = w_ref[...]
        wr_ref[...] = (w32 * (-2.0 * rc)[None, :]).astype(jnp.bfloat16)
        w2_ref[0, :] = jnp.sum(w32 * w32 * rc[None, :], axis=1)

    x32 = x_ref[...]
    xb = x32.astype(jnp.bfloat16)
    cross = jax.lax.dot_general(
        xb, wr_ref[...], (((1,), (1,)), ((), ())),
        preferred_element_type=jnp.float32)
    x2 = jnp.sum(x32 * x32 * rc[None, :], axis=1)
    out_ref[...] = cross + x2[:, None] + w2_ref[0, :][None, :]


def kernel(X, W, r):
    n, f = X.shape
    c, _ = W.shape
    rc = jnp.clip(r, 1e-6, None).reshape(1, f)
    return pl.pallas_call(
        _gmlvq_body,
        grid=(n // BN,),
        in_specs=[
            pl.BlockSpec((BN, f), lambda i: (i, 0)),
            pl.BlockSpec((c, f), lambda i: (0, 0)),
            pl.BlockSpec((1, f), lambda i: (0, 0)),
        ],
        out_specs=pl.BlockSpec((BN, c), lambda i: (i, 0)),
        out_shape=jax.ShapeDtypeStruct((n, c), jnp.float32),
        scratch_shapes=[
            pltpu.VMEM((c, f), jnp.bfloat16),
            pltpu.VMEM((1, c), jnp.float32),
        ],
        compiler_params=pltpu.CompilerParams(
            dimension_semantics=("arbitrary",)),
    )(X, W, rc)
